# baseline (speedup 1.0000x reference)
import functools

import jax
import jax.numpy as jnp
from jax import lax
from jax.experimental import pallas as pl
from jax.experimental.pallas import tpu as pltpu
from jax.experimental.pallas import tpu_sc as plsc

B = 4096
F = 1000
D = 64
K = 20
REG = 0.01

_NW = 32
_BPW = B // _NW

_R = 512
_G = B // _R
_RC = 1024
_GC = B // _RC


_NS = 51200
_TC = 1024
_TG = _NS // _TC
_TLAST = -(-100000 // _TC) - 1


def _transpose_pair_body(lo_ref, hi_ref, out_ref):
    out_ref[...] = jnp.concatenate(
        [lo_ref[...].T, hi_ref[...].T], axis=1)


def _tc_transpose_pair(Ut, interpret=False):
    return pl.pallas_call(
        _transpose_pair_body,
        grid=(_TG,),
        in_specs=[
            pl.BlockSpec((D, _TC), lambda i: (0, i)),
            pl.BlockSpec((D, _TC),
                         lambda i: (0, jnp.minimum(i + _TG, _TLAST))),
        ],
        out_specs=pl.BlockSpec((_TC, 2 * D), lambda i: (i, 0)),
        out_shape=jax.ShapeDtypeStruct((_NS, 2 * D), jnp.float32),
        interpret=interpret,
    )(Ut, Ut)


def _sc_gather(Uwide, Vwide, uid, pid, nid):
    mesh = plsc.VectorSubcoreMesh(core_axis_name="c", subcore_axis_name="s")

    @functools.partial(
        pl.kernel,
        mesh=mesh,
        compiler_params=pltpu.CompilerParams(use_tc_tiling_on_sc=True),
        out_type=[jax.ShapeDtypeStruct((B, 2 * D), jnp.float32)] * 3,
        scratch_types=[
            pltpu.VMEM((_BPW,), jnp.int32),
            pltpu.VMEM((_BPW,), jnp.int32),
            pltpu.VMEM((_BPW,), jnp.int32),
            pltpu.VMEM((_BPW, 2 * D), jnp.float32),
            pltpu.VMEM((_BPW, 2 * D), jnp.float32),
            pltpu.VMEM((_BPW, 2 * D), jnp.float32),
            pltpu.SemaphoreType.DMA,
            pltpu.SemaphoreType.DMA,
            pltpu.SemaphoreType.DMA,
        ],
    )
    def gather_k(u_hbm, v_hbm, uid_hbm, pid_hbm, nid_hbm, ou, op, on,
                 iu, ip, inn, ru, rp, rn, su, sp, sn):
        wid = lax.axis_index("s") * 2 + lax.axis_index("c")
        base = wid * _BPW
        pltpu.sync_copy(uid_hbm.at[pl.ds(base, _BPW)], iu)
        pltpu.sync_copy(pid_hbm.at[pl.ds(base, _BPW)], ip)
        pltpu.sync_copy(nid_hbm.at[pl.ds(base, _BPW)], inn)
        cu = pltpu.async_copy(u_hbm.at[iu], ru, su)
        cp = pltpu.async_copy(v_hbm.at[ip], rp, sp)
        cn = pltpu.async_copy(v_hbm.at[inn], rn, sn)
        cu.wait()
        cp.wait()
        cn.wait()
        pltpu.sync_copy(ru, ou.at[pl.ds(base, _BPW)])
        pltpu.sync_copy(rp, op.at[pl.ds(base, _BPW)])
        pltpu.sync_copy(rn, on.at[pl.ds(base, _BPW)])

    return gather_k(Uwide, Vwide, uid, pid, nid)


def _heavy_body(x_ref, tau_ref, pif_ref, nif_ref, wut_ref, wit_ref,
                ufwu_ref, pwi_ref, nwi_ref, reg_ref, acc_ref):
    i = pl.program_id(0)
    x = x_ref[...]

    rows = lax.broadcasted_iota(jnp.int32, (F, _R), 0)
    work = x
    for _ in range(K):
        r = jnp.argmax(work, axis=0)
        work = jnp.where(rows == r[None, :], -jnp.inf, work)

    mtau = jnp.where(work == -jnp.inf, tau_ref[...], 0.0)
    uf = x + mtau
    dims = (((1,), (0,)), ((), ()))
    ufwu_ref[...] = lax.dot_general(
        wut_ref[...], uf, dims, preferred_element_type=jnp.float32).T
    pwi_ref[...] = lax.dot_general(
        wit_ref[...], pif_ref[...], dims,
        preferred_element_type=jnp.float32).T
    nwi_ref[...] = lax.dot_general(
        wit_ref[...], nif_ref[...], dims,
        preferred_element_type=jnp.float32).T

    @pl.when(i == 0)
    def _init():
        acc_ref[0] = 0.0

    acc_ref[0] += jnp.sum(mtau * mtau)

    @pl.when(i == _G - 1)
    def _fin():
        reg_ref[0, 0] = acc_ref[0]


def _combine_body(ufwu_ref, pwi_ref, nwi_ref, ug_ref, vp_ref, vn_ref,
                  paru_ref, parp_ref, parn_ref, reg_ref,
                  conf_ref, loss_ref, acc_ref):
    i = pl.program_id(0)

    def half(g_ref, par_ref):
        g = g_ref[...]
        return jnp.where(par_ref[...] > 0, g[:, D:], g[:, :D])

    ue = half(ug_ref, paru_ref) + ufwu_ref[...]
    pos = jnp.sum(ue * (half(vp_ref, parp_ref) + pwi_ref[...]), axis=1)
    neg = jnp.sum(ue * (half(vn_ref, parn_ref) + nwi_ref[...]), axis=1)
    d = pos - neg
    conf = jnp.maximum(d, 0.0) + jnp.log1p(jnp.exp(-jnp.abs(d)))
    conf_ref[0, 0, :] = conf

    @pl.when(i == 0)
    def _init():
        acc_ref[0] = 0.0

    acc_ref[0] += jnp.sum(conf)

    @pl.when(i == _GC - 1)
    def _fin():
        loss_ref[0, 0] = acc_ref[0] + REG * jnp.sqrt(reg_ref[0, 0])


def _tc_heavy(ufb_t, tau_t, pif_t, nif_t, WuT, WiT, interpret=False):
    col_spec = pl.BlockSpec((F, _R), lambda i: (0, i))
    w_spec = pl.BlockSpec((D, F), lambda i: (0, 0))
    emb_spec = pl.BlockSpec((_R, D), lambda i: (i, 0))
    return pl.pallas_call(
        _heavy_body,
        grid=(_G,),
        in_specs=[col_spec, col_spec, col_spec, col_spec, w_spec, w_spec],
        out_specs=[
            emb_spec, emb_spec, emb_spec,
            pl.BlockSpec((1, 1), lambda i: (0, 0), memory_space=pltpu.SMEM),
        ],
        out_shape=[
            jax.ShapeDtypeStruct((B, D), jnp.float32),
            jax.ShapeDtypeStruct((B, D), jnp.float32),
            jax.ShapeDtypeStruct((B, D), jnp.float32),
            jax.ShapeDtypeStruct((1, 1), jnp.float32),
        ],
        scratch_shapes=[pltpu.SMEM((1,), jnp.float32)],
        interpret=interpret,
    )(ufb_t, tau_t, pif_t, nif_t, WuT, WiT)


def _tc_combine(ufwu, pwi, nwi, Ug, Vp, Vn, paru, parp, parn, regsum,
                interpret=False):
    emb_spec = pl.BlockSpec((_RC, D), lambda i: (i, 0))
    g_spec = pl.BlockSpec((_RC, 2 * D), lambda i: (i, 0))
    par_spec = pl.BlockSpec((_RC, 1), lambda i: (i, 0))
    return pl.pallas_call(
        _combine_body,
        grid=(_GC,),
        in_specs=[
            emb_spec, emb_spec, emb_spec, g_spec, g_spec, g_spec,
            par_spec, par_spec, par_spec,
            pl.BlockSpec((1, 1), lambda i: (0, 0), memory_space=pltpu.SMEM),
        ],
        out_specs=[
            pl.BlockSpec((1, 1, _RC), lambda i: (i, 0, 0)),
            pl.BlockSpec((1, 1), lambda i: (0, 0), memory_space=pltpu.SMEM),
        ],
        out_shape=[
            jax.ShapeDtypeStruct((_GC, 1, _RC), jnp.float32),
            jax.ShapeDtypeStruct((1, 1), jnp.float32),
        ],
        scratch_shapes=[pltpu.SMEM((1,), jnp.float32)],
        interpret=interpret,
    )(ufwu, pwi, nwi, Ug, Vp, Vn, paru, parp, parn, regsum)


def kernel(user_batch, user_feature_batch, pos_item_batch,
           pos_item_feature_batch, neg_item_batch, neg_item_feature_batch,
           tau, U, V, Wu, Wi):
    uid = user_batch.astype(jnp.int32)
    pid = pos_item_batch.astype(jnp.int32)
    nid = neg_item_batch.astype(jnp.int32)
    Uwide = _tc_transpose_pair(U.T)
    Vwide = _tc_transpose_pair(V.T)
    Ug, Vp, Vn = _sc_gather(Uwide, Vwide,
                            uid % _NS, pid % _NS, nid % _NS)
    paru = (uid >= _NS).astype(jnp.float32)[:, None]
    parp = (pid >= _NS).astype(jnp.float32)[:, None]
    parn = (nid >= _NS).astype(jnp.float32)[:, None]
    ufwu, pwi, nwi, regsum = _tc_heavy(
        user_feature_batch.T, tau.T, pos_item_feature_batch.T,
        neg_item_feature_batch.T, Wu.T, Wi.T)
    conf2d, loss = _tc_combine(ufwu, pwi, nwi, Ug, Vp, Vn,
                               paru, parp, parn, regsum)
    return (loss[0, 0], conf2d.reshape(B))


# speedup vs baseline: 1.0658x; 1.0658x over previous
import functools

import jax
import jax.numpy as jnp
from jax import lax
from jax.experimental import pallas as pl
from jax.experimental.pallas import tpu as pltpu
from jax.experimental.pallas import tpu_sc as plsc

B = 4096
F = 1000
D = 64
K = 20
REG = 0.01

_NW = 32
_BPW = B // _NW

_R = 512
_G = B // _R
_RC = 1024
_GC = B // _RC


_TC = 2048
_TG = -(-100000 // _TC)


def _transpose_pad_body(ut_ref, out_ref):
    t = ut_ref[...].T
    out_ref[...] = jnp.concatenate(
        [t, jnp.zeros((_TC, D), jnp.float32)], axis=1)


def _tc_transpose_pad(Ut, interpret=False):
    n = Ut.shape[1]
    return pl.pallas_call(
        _transpose_pad_body,
        grid=(_TG,),
        in_specs=[pl.BlockSpec((D, _TC), lambda i: (0, i))],
        out_specs=pl.BlockSpec((_TC, 2 * D), lambda i: (i, 0)),
        out_shape=jax.ShapeDtypeStruct((n, 2 * D), jnp.float32),
        interpret=interpret,
    )(Ut)


def _sc_gather(Uwide, Vwide, uid, pid, nid):
    mesh = plsc.VectorSubcoreMesh(core_axis_name="c", subcore_axis_name="s")

    @functools.partial(
        pl.kernel,
        mesh=mesh,
        compiler_params=pltpu.CompilerParams(use_tc_tiling_on_sc=True),
        out_type=[jax.ShapeDtypeStruct((B, 2 * D), jnp.float32)] * 3,
        scratch_types=[
            pltpu.VMEM((_BPW,), jnp.int32),
            pltpu.VMEM((_BPW,), jnp.int32),
            pltpu.VMEM((_BPW,), jnp.int32),
            pltpu.VMEM((_BPW, 2 * D), jnp.float32),
            pltpu.VMEM((_BPW, 2 * D), jnp.float32),
            pltpu.VMEM((_BPW, 2 * D), jnp.float32),
            pltpu.SemaphoreType.DMA,
            pltpu.SemaphoreType.DMA,
            pltpu.SemaphoreType.DMA,
        ],
    )
    def gather_k(u_hbm, v_hbm, uid_hbm, pid_hbm, nid_hbm, ou, op, on,
                 iu, ip, inn, ru, rp, rn, su, sp, sn):
        wid = lax.axis_index("s") * 2 + lax.axis_index("c")
        base = wid * _BPW
        pltpu.sync_copy(uid_hbm.at[pl.ds(base, _BPW)], iu)
        pltpu.sync_copy(pid_hbm.at[pl.ds(base, _BPW)], ip)
        pltpu.sync_copy(nid_hbm.at[pl.ds(base, _BPW)], inn)
        cu = pltpu.async_copy(u_hbm.at[iu], ru, su)
        cp = pltpu.async_copy(v_hbm.at[ip], rp, sp)
        cn = pltpu.async_copy(v_hbm.at[inn], rn, sn)
        cu.wait()
        cp.wait()
        cn.wait()
        pltpu.sync_copy(ru, ou.at[pl.ds(base, _BPW)])
        pltpu.sync_copy(rp, op.at[pl.ds(base, _BPW)])
        pltpu.sync_copy(rn, on.at[pl.ds(base, _BPW)])

    return gather_k(Uwide, Vwide, uid, pid, nid)


def _heavy_body(x_ref, tau_ref, pif_ref, nif_ref, wut_ref, wit_ref,
                ufwu_ref, pwi_ref, nwi_ref, reg_ref, acc_ref):
    i = pl.program_id(0)
    x = x_ref[...]

    rows = lax.broadcasted_iota(jnp.int32, (F, _R), 0)
    work = x
    for _ in range(K):
        r = jnp.argmax(work, axis=0)
        work = jnp.where(rows == r[None, :], -jnp.inf, work)

    mtau = jnp.where(work == -jnp.inf, tau_ref[...], 0.0)
    uf = x + mtau
    dims = (((1,), (0,)), ((), ()))
    ufwu_ref[...] = lax.dot_general(
        wut_ref[...], uf, dims, preferred_element_type=jnp.float32).T
    pwi_ref[...] = lax.dot_general(
        wit_ref[...], pif_ref[...], dims,
        preferred_element_type=jnp.float32).T
    nwi_ref[...] = lax.dot_general(
        wit_ref[...], nif_ref[...], dims,
        preferred_element_type=jnp.float32).T

    @pl.when(i == 0)
    def _init():
        acc_ref[0] = 0.0

    acc_ref[0] += jnp.sum(mtau * mtau)

    @pl.when(i == _G - 1)
    def _fin():
        reg_ref[0, 0] = acc_ref[0]


def _combine_body(ufwu_ref, pwi_ref, nwi_ref, ug_ref, vp_ref, vn_ref,
                  reg_ref, conf_ref, loss_ref, acc_ref):
    i = pl.program_id(0)
    ue = ug_ref[:, :D] + ufwu_ref[...]
    pos = jnp.sum(ue * (vp_ref[:, :D] + pwi_ref[...]), axis=1)
    neg = jnp.sum(ue * (vn_ref[:, :D] + nwi_ref[...]), axis=1)
    d = pos - neg
    conf = jnp.maximum(d, 0.0) + jnp.log1p(jnp.exp(-jnp.abs(d)))
    conf_ref[0, 0, :] = conf

    @pl.when(i == 0)
    def _init():
        acc_ref[0] = 0.0

    acc_ref[0] += jnp.sum(conf)

    @pl.when(i == _GC - 1)
    def _fin():
        loss_ref[0, 0] = acc_ref[0] + REG * jnp.sqrt(reg_ref[0, 0])


def _tc_heavy(ufb_t, tau_t, pif_t, nif_t, WuT, WiT, interpret=False):
    col_spec = pl.BlockSpec((F, _R), lambda i: (0, i))
    w_spec = pl.BlockSpec((D, F), lambda i: (0, 0))
    emb_spec = pl.BlockSpec((_R, D), lambda i: (i, 0))
    return pl.pallas_call(
        _heavy_body,
        grid=(_G,),
        in_specs=[col_spec, col_spec, col_spec, col_spec, w_spec, w_spec],
        out_specs=[
            emb_spec, emb_spec, emb_spec,
            pl.BlockSpec((1, 1), lambda i: (0, 0), memory_space=pltpu.SMEM),
        ],
        out_shape=[
            jax.ShapeDtypeStruct((B, D), jnp.float32),
            jax.ShapeDtypeStruct((B, D), jnp.float32),
            jax.ShapeDtypeStruct((B, D), jnp.float32),
            jax.ShapeDtypeStruct((1, 1), jnp.float32),
        ],
        scratch_shapes=[pltpu.SMEM((1,), jnp.float32)],
        interpret=interpret,
    )(ufb_t, tau_t, pif_t, nif_t, WuT, WiT)


def _tc_combine(ufwu, pwi, nwi, Ug, Vp, Vn, regsum, interpret=False):
    emb_spec = pl.BlockSpec((_RC, D), lambda i: (i, 0))
    g_spec = pl.BlockSpec((_RC, 2 * D), lambda i: (i, 0))
    return pl.pallas_call(
        _combine_body,
        grid=(_GC,),
        in_specs=[
            emb_spec, emb_spec, emb_spec, g_spec, g_spec, g_spec,
            pl.BlockSpec((1, 1), lambda i: (0, 0), memory_space=pltpu.SMEM),
        ],
        out_specs=[
            pl.BlockSpec((1, 1, _RC), lambda i: (i, 0, 0)),
            pl.BlockSpec((1, 1), lambda i: (0, 0), memory_space=pltpu.SMEM),
        ],
        out_shape=[
            jax.ShapeDtypeStruct((_GC, 1, _RC), jnp.float32),
            jax.ShapeDtypeStruct((1, 1), jnp.float32),
        ],
        scratch_shapes=[pltpu.SMEM((1,), jnp.float32)],
        interpret=interpret,
    )(ufwu, pwi, nwi, Ug, Vp, Vn, regsum)


def kernel(user_batch, user_feature_batch, pos_item_batch,
           pos_item_feature_batch, neg_item_batch, neg_item_feature_batch,
           tau, U, V, Wu, Wi):
    uid = user_batch.astype(jnp.int32)
    pid = pos_item_batch.astype(jnp.int32)
    nid = neg_item_batch.astype(jnp.int32)
    Uwide = _tc_transpose_pad(U.T)
    Vwide = _tc_transpose_pad(V.T)
    Ug, Vp, Vn = _sc_gather(Uwide, Vwide, uid, pid, nid)
    ufwu, pwi, nwi, regsum = _tc_heavy(
        user_feature_batch.T, tau.T, pos_item_feature_batch.T,
        neg_item_feature_batch.T, Wu.T, Wi.T)
    conf2d, loss = _tc_combine(ufwu, pwi, nwi, Ug, Vp, Vn, regsum)
    return (loss[0, 0], conf2d.reshape(B))


# speedup vs baseline: 1.0680x; 1.0021x over previous
import functools

import jax
import jax.numpy as jnp
from jax import lax
from jax.experimental import pallas as pl
from jax.experimental.pallas import tpu as pltpu
from jax.experimental.pallas import tpu_sc as plsc

B = 4096
F = 1000
D = 64
K = 20
REG = 0.01

_NW = 32
_BPW = B // _NW

_R = 512
_G = B // _R
_RC = 1024
_GC = B // _RC


_TC = 2048
_TG = -(-100000 // _TC)


def _transpose_pad_body(ut_ref, out_ref):
    t = ut_ref[...].T
    out_ref[...] = jnp.concatenate(
        [t, jnp.zeros((_TC, D), jnp.float32)], axis=1)


def _tc_transpose_pad(Ut):
    n = Ut.shape[1]
    return pl.pallas_call(
        _transpose_pad_body,
        grid=(_TG,),
        in_specs=[pl.BlockSpec((D, _TC), lambda i: (0, i))],
        out_specs=pl.BlockSpec((_TC, 2 * D), lambda i: (i, 0)),
        out_shape=jax.ShapeDtypeStruct((n, 2 * D), jnp.float32),
    )(Ut)


def _sc_gather(Uwide, Vwide, uid, pid, nid):
    mesh = plsc.VectorSubcoreMesh(core_axis_name="c", subcore_axis_name="s")

    @functools.partial(
        pl.kernel,
        mesh=mesh,
        compiler_params=pltpu.CompilerParams(use_tc_tiling_on_sc=True),
        out_type=[jax.ShapeDtypeStruct((B, 2 * D), jnp.float32)] * 3,
        scratch_types=[
            pltpu.VMEM((_BPW,), jnp.int32),
            pltpu.VMEM((_BPW,), jnp.int32),
            pltpu.VMEM((_BPW,), jnp.int32),
            pltpu.VMEM((_BPW, 2 * D), jnp.float32),
            pltpu.VMEM((_BPW, 2 * D), jnp.float32),
            pltpu.VMEM((_BPW, 2 * D), jnp.float32),
            pltpu.SemaphoreType.DMA,
            pltpu.SemaphoreType.DMA,
            pltpu.SemaphoreType.DMA,
        ],
    )
    def gather_k(u_hbm, v_hbm, uid_hbm, pid_hbm, nid_hbm, ou, op, on,
                 iu, ip, inn, ru, rp, rn, su, sp, sn):
        wid = lax.axis_index("s") * 2 + lax.axis_index("c")
        base = wid * _BPW
        pltpu.sync_copy(uid_hbm.at[pl.ds(base, _BPW)], iu)
        pltpu.sync_copy(pid_hbm.at[pl.ds(base, _BPW)], ip)
        pltpu.sync_copy(nid_hbm.at[pl.ds(base, _BPW)], inn)
        cu = pltpu.async_copy(u_hbm.at[iu], ru, su)
        cp = pltpu.async_copy(v_hbm.at[ip], rp, sp)
        cn = pltpu.async_copy(v_hbm.at[inn], rn, sn)
        cu.wait()
        cp.wait()
        cn.wait()
        pltpu.sync_copy(ru, ou.at[pl.ds(base, _BPW)])
        pltpu.sync_copy(rp, op.at[pl.ds(base, _BPW)])
        pltpu.sync_copy(rn, on.at[pl.ds(base, _BPW)])

    return gather_k(Uwide, Vwide, uid, pid, nid)


def _heavy_body(x_ref, tau_ref, pif_ref, nif_ref, wut_ref, wit_ref,
                ufwu_ref, pwi_ref, nwi_ref, reg_ref, acc_ref):
    i = pl.program_id(0)
    x = x_ref[...]

    rows = lax.broadcasted_iota(jnp.int32, (F, _R), 0)
    work = x
    for _ in range(K):
        r = jnp.argmax(work, axis=0)
        work = jnp.where(rows == r[None, :], -jnp.inf, work)

    mtau = jnp.where(work == -jnp.inf, tau_ref[...], 0.0)
    uf = x + mtau
    dims = (((1,), (0,)), ((), ()))
    ufwu_ref[...] = lax.dot_general(
        wut_ref[...], uf, dims, preferred_element_type=jnp.float32).T
    pwi_ref[...] = lax.dot_general(
        wit_ref[...], pif_ref[...], dims,
        preferred_element_type=jnp.float32).T
    nwi_ref[...] = lax.dot_general(
        wit_ref[...], nif_ref[...], dims,
        preferred_element_type=jnp.float32).T

    @pl.when(i == 0)
    def _init():
        acc_ref[0] = 0.0

    acc_ref[0] += jnp.sum(mtau * mtau)

    @pl.when(i == _G - 1)
    def _fin():
        reg_ref[0, 0] = acc_ref[0]


def _combine_body(ufwu_ref, pwi_ref, nwi_ref, ug_ref, vp_ref, vn_ref,
                  reg_ref, conf_ref, loss_ref, acc_ref):
    i = pl.program_id(0)
    ue = ug_ref[:, :D] + ufwu_ref[...]
    pos = jnp.sum(ue * (vp_ref[:, :D] + pwi_ref[...]), axis=1)
    neg = jnp.sum(ue * (vn_ref[:, :D] + nwi_ref[...]), axis=1)
    d = pos - neg
    conf = jnp.maximum(d, 0.0) + jnp.log1p(jnp.exp(-jnp.abs(d)))
    conf_ref[0, 0, :] = conf

    @pl.when(i == 0)
    def _init():
        acc_ref[0] = 0.0

    acc_ref[0] += jnp.sum(conf)

    @pl.when(i == _GC - 1)
    def _fin():
        loss_ref[0, 0] = acc_ref[0] + REG * jnp.sqrt(reg_ref[0, 0])


def _tc_heavy(ufb_t, tau_t, pif_t, nif_t, WuT, WiT):
    col_spec = pl.BlockSpec((F, _R), lambda i: (0, i))
    w_spec = pl.BlockSpec((D, F), lambda i: (0, 0))
    emb_spec = pl.BlockSpec((_R, D), lambda i: (i, 0))
    return pl.pallas_call(
        _heavy_body,
        grid=(_G,),
        in_specs=[col_spec, col_spec, col_spec, col_spec, w_spec, w_spec],
        out_specs=[
            emb_spec, emb_spec, emb_spec,
            pl.BlockSpec((1, 1), lambda i: (0, 0), memory_space=pltpu.SMEM),
        ],
        out_shape=[
            jax.ShapeDtypeStruct((B, D), jnp.float32),
            jax.ShapeDtypeStruct((B, D), jnp.float32),
            jax.ShapeDtypeStruct((B, D), jnp.float32),
            jax.ShapeDtypeStruct((1, 1), jnp.float32),
        ],
        scratch_shapes=[pltpu.SMEM((1,), jnp.float32)],
    )(ufb_t, tau_t, pif_t, nif_t, WuT, WiT)


def _tc_combine(ufwu, pwi, nwi, Ug, Vp, Vn, regsum):
    emb_spec = pl.BlockSpec((_RC, D), lambda i: (i, 0))
    g_spec = pl.BlockSpec((_RC, 2 * D), lambda i: (i, 0))
    return pl.pallas_call(
        _combine_body,
        grid=(_GC,),
        in_specs=[
            emb_spec, emb_spec, emb_spec, g_spec, g_spec, g_spec,
            pl.BlockSpec((1, 1), lambda i: (0, 0), memory_space=pltpu.SMEM),
        ],
        out_specs=[
            pl.BlockSpec((1, 1, _RC), lambda i: (i, 0, 0)),
            pl.BlockSpec((1, 1), lambda i: (0, 0), memory_space=pltpu.SMEM),
        ],
        out_shape=[
            jax.ShapeDtypeStruct((_GC, 1, _RC), jnp.float32),
            jax.ShapeDtypeStruct((1, 1), jnp.float32),
        ],
        scratch_shapes=[pltpu.SMEM((1,), jnp.float32)],
    )(ufwu, pwi, nwi, Ug, Vp, Vn, regsum)


def kernel(user_batch, user_feature_batch, pos_item_batch,
           pos_item_feature_batch, neg_item_batch, neg_item_feature_batch,
           tau, U, V, Wu, Wi):
    uid = user_batch.astype(jnp.int32)
    pid = pos_item_batch.astype(jnp.int32)
    nid = neg_item_batch.astype(jnp.int32)
    Uwide = _tc_transpose_pad(U.T)
    Vwide = _tc_transpose_pad(V.T)
    Ug, Vp, Vn = _sc_gather(Uwide, Vwide, uid, pid, nid)
    ufwu, pwi, nwi, regsum = _tc_heavy(
        user_feature_batch.T, tau.T, pos_item_feature_batch.T,
        neg_item_feature_batch.T, Wu.T, Wi.T)
    conf2d, loss = _tc_combine(ufwu, pwi, nwi, Ug, Vp, Vn, regsum)
    return (loss[0, 0], conf2d.reshape(B))
